# Initial kernel scaffold; baseline (speedup 1.0000x reference)
import jax
import jax.numpy as jnp
from jax.experimental import pallas as pl


def kernel(center_words, target_words, all_vocabs, emb_v, emb_u):
    raise NotImplementedError("write your pallas kernel here")


# speedup vs baseline: 51.8004x; 51.8004x over previous
import functools

import jax
import jax.numpy as jnp
from jax import lax
from jax.experimental import pallas as pl
from jax.experimental.pallas import tpu as pltpu
from jax.experimental.pallas import tpu_sc as plsc

VOCAB = 1000
EMB = 64
B = 1024

NC = 2
NS = 16
NW = NC * NS
ROWS_PER_W = B // NW
G = 16
GROUPS = ROWS_PER_W // G


def _tc_scores_body(cw_ref, tw_ref, emb_v_ref, emb_u_ref, s_ref, smean_ref):
    cw = cw_ref[...]
    tw = tw_ref[...]
    iota_v = lax.broadcasted_iota(jnp.int32, (B, VOCAB), 1)
    onehot_c = (cw == iota_v).astype(jnp.float32)
    onehot_t = (tw == iota_v).astype(jnp.float32)
    emb_v = emb_v_ref[...]
    emb_u = emb_u_ref[...]
    c = jnp.dot(onehot_c, emb_v, preferred_element_type=jnp.float32,
                precision=lax.Precision.HIGHEST)
    t = jnp.dot(onehot_t, emb_u, preferred_element_type=jnp.float32,
                precision=lax.Precision.HIGHEST)
    s = lax.dot_general(c, emb_u, (((1,), (1,)), ((), ())),
                        preferred_element_type=jnp.float32,
                        precision=lax.Precision.HIGHEST)
    s_ref[...] = s
    smean_ref[...] = (jnp.sum(c * t) / B).reshape(1, 1)


def _sc_sumexp_body(s_hbm, av_hbm, out_hbm, s_v, av_v, out_v):
    cid = lax.axis_index("c")
    sid = lax.axis_index("s")
    wid = sid * NC + cid
    base = wid * ROWS_PER_W
    lanes = lax.broadcasted_iota(jnp.int32, (G,), 0)
    for g in range(GROUPS):
        row0 = base + g * G
        pltpu.sync_copy(s_hbm.at[pl.ds(row0, G)], s_v)
        pltpu.sync_copy(av_hbm.at[pl.ds(row0, G)], av_v)

        def step(j, acc):
            col = jnp.full((G,), j, dtype=jnp.int32)
            iv = plsc.load_gather(av_v, [lanes, col])
            vals = plsc.load_gather(s_v, [lanes, iv])
            return acc + jnp.exp(vals)

        acc = lax.fori_loop(0, VOCAB, step, jnp.zeros((G,), jnp.float32))
        out_v[pl.ds(g * G, G)] = acc
    pltpu.sync_copy(out_v, out_hbm.at[pl.ds(base, ROWS_PER_W)])


def _tc_final_body(sumexp_ref, smean_ref, o_ref):
    o_ref[...] = jnp.mean(jnp.log(sumexp_ref[...])).reshape(1, 1) - smean_ref[...]


@jax.jit
def kernel(center_words, target_words, all_vocabs, emb_v, emb_u):
    s, smean = pl.pallas_call(
        _tc_scores_body,
        out_shape=[
            jax.ShapeDtypeStruct((B, VOCAB), jnp.float32),
            jax.ShapeDtypeStruct((1, 1), jnp.float32),
        ],
    )(center_words, target_words, emb_v, emb_u)

    sumexp = pl.kernel(
        _sc_sumexp_body,
        mesh=plsc.VectorSubcoreMesh(core_axis_name="c", subcore_axis_name="s"),
        out_type=jax.ShapeDtypeStruct((B,), jnp.float32),
        scratch_types=[
            pltpu.VMEM((G, VOCAB), jnp.float32),
            pltpu.VMEM((G, VOCAB), jnp.int32),
            pltpu.VMEM((ROWS_PER_W,), jnp.float32),
        ],
        compiler_params=pltpu.CompilerParams(
            use_tc_tiling_on_sc=False, needs_layout_passes=False),
    )(s, all_vocabs)

    nll = pl.pallas_call(
        _tc_final_body,
        out_shape=jax.ShapeDtypeStruct((1, 1), jnp.float32),
    )(sumexp.reshape(8, 128), smean)
    return nll[0, 0]


# speedup vs baseline: 71.7443x; 1.3850x over previous
import functools

import jax
import jax.numpy as jnp
from jax import lax
from jax.experimental import pallas as pl
from jax.experimental.pallas import tpu as pltpu
from jax.experimental.pallas import tpu_sc as plsc

VOCAB = 1000
EMB = 64
B = 1024
VPAD = 1024

NC = 2
NS = 16
NW = NC * NS
RPW = B // NW
G = 16
GROUPS = RPW // G
UNROLL = 8


def _tc_p_body(emb_v_ref, emb_u_ref, p_ref):
    p_ref[...] = lax.dot_general(
        emb_v_ref[...], emb_u_ref[...], (((1,), (1,)), ((), ())),
        preferred_element_type=jnp.float32, precision=lax.Precision.HIGHEST)


def _sc_sumexp_body(p_hbm, cen_hbm, tgt_hbm, av_hbm, se_hbm, sco_hbm,
                    cen_v, tgt_v, rows_v, av_v, se_v, sco_v, sem_r, sem_a):
    cid = lax.axis_index("c")
    sid = lax.axis_index("s")
    wid = sid * NC + cid
    base = wid * RPW
    lanes = lax.broadcasted_iota(jnp.int32, (G,), 0)

    pltpu.sync_copy(cen_hbm.at[pl.ds(base, RPW)], cen_v)
    pltpu.sync_copy(tgt_hbm.at[pl.ds(base, RPW)], tgt_v)
    cp_rows = pltpu.async_copy(p_hbm.at[cen_v], rows_v, sem_r)
    cp_av = pltpu.async_copy(av_hbm.at[pl.ds(base, RPW)], av_v, sem_a)
    cp_rows.wait()
    cp_av.wait()

    def step(j0, accs):
        new = list(accs)
        for u in range(UNROLL):
            col = jnp.full((G,), j0 * UNROLL + u, dtype=jnp.int32)
            for g in range(GROUPS):
                gl = lanes + g * G
                iv = plsc.load_gather(av_v, [gl, col])
                vals = plsc.load_gather(rows_v, [gl, iv])
                new[g] = new[g] + jnp.exp(vals)
        return tuple(new)

    zero = jnp.zeros((G,), jnp.float32)
    accs = lax.fori_loop(0, VOCAB // UNROLL, step, (zero,) * GROUPS)

    for g in range(GROUPS):
        se_v[pl.ds(g * G, G)] = accs[g]
        gl = lanes + g * G
        tv = tgt_v[pl.ds(g * G, G)]
        sco_v[pl.ds(g * G, G)] = plsc.load_gather(rows_v, [gl, tv])
    pltpu.sync_copy(se_v, se_hbm.at[pl.ds(base, RPW)])
    pltpu.sync_copy(sco_v, sco_hbm.at[pl.ds(base, RPW)])


def _tc_final_body(sumexp_ref, scores_ref, o_ref):
    nll = jnp.mean(jnp.log(sumexp_ref[...])) - jnp.mean(scores_ref[...])
    o_ref[...] = nll.reshape(1, 1)


@jax.jit
def kernel(center_words, target_words, all_vocabs, emb_v, emb_u):
    emb_u_pad = jnp.pad(emb_u, ((0, VPAD - VOCAB), (0, 0)))

    p = pl.pallas_call(
        _tc_p_body,
        out_shape=jax.ShapeDtypeStruct((VOCAB, VPAD), jnp.float32),
    )(emb_v, emb_u_pad)

    sumexp, scores = pl.kernel(
        _sc_sumexp_body,
        mesh=plsc.VectorSubcoreMesh(core_axis_name="c", subcore_axis_name="s"),
        out_type=[
            jax.ShapeDtypeStruct((B,), jnp.float32),
            jax.ShapeDtypeStruct((B,), jnp.float32),
        ],
        scratch_types=[
            pltpu.VMEM((RPW,), jnp.int32),
            pltpu.VMEM((RPW,), jnp.int32),
            pltpu.VMEM((RPW, VPAD), jnp.float32),
            pltpu.VMEM((RPW, VOCAB), jnp.int32),
            pltpu.VMEM((RPW,), jnp.float32),
            pltpu.VMEM((RPW,), jnp.float32),
            pltpu.SemaphoreType.DMA,
            pltpu.SemaphoreType.DMA,
        ],
        compiler_params=pltpu.CompilerParams(
            use_tc_tiling_on_sc=False, needs_layout_passes=False),
    )(p, center_words.reshape(B), target_words.reshape(B), all_vocabs)

    nll = pl.pallas_call(
        _tc_final_body,
        out_shape=jax.ShapeDtypeStruct((1, 1), jnp.float32),
    )(sumexp.reshape(8, 128), scores.reshape(8, 128))
    return nll[0, 0]
